# Initial kernel scaffold; baseline (speedup 1.0000x reference)
import jax
import jax.numpy as jnp
from jax.experimental import pallas as pl


def kernel(x, edge_index, batch, W1l, b1l, W1r, b1r, a1, c1, g1, be1, W2l, b2l, W2r, b2r, a2, c2, g2, be2, Wres, bres):
    raise NotImplementedError("write your pallas kernel here")


# speedup vs baseline: 4.9447x; 4.9447x over previous
import functools

import jax
import jax.numpy as jnp
from jax import lax
from jax.experimental import pallas as pl
from jax.experimental.pallas import tpu as pltpu
from jax.experimental.pallas import tpu_sc as plsc

_N = 28672
_B = 2048
_NPG = 14
_E0 = 458752
_ET = _E0 + _N
_D1 = 128
_D2 = 256
_CW = 64
_NC = 2
_NS = 16
_NW = _NC * _NS
_T = 128
_EW = _ET // _NW
_EP = _ET // _NS
_NB1 = _EW // _T
_NB2 = _EP // _T
_RPT = _N // _NS


def _sc_mesh():
    return plsc.VectorSubcoreMesh(core_axis_name="c", subcore_axis_name="s")


_SC_PARAMS = pltpu.CompilerParams(needs_layout_passes=False)
_NH = _N // 2


def _build_p1(D):
    @functools.partial(
        pl.kernel,
        out_type=(jax.ShapeDtypeStruct((_ET,), jnp.float32),
                  jax.ShapeDtypeStruct((_NW, 16), jnp.float32)),
        mesh=_sc_mesh(),
        compiler_params=_SC_PARAMS,
        scratch_types=[
            pltpu.VMEM((_T,), jnp.int32),
            pltpu.VMEM((_T,), jnp.int32),
            pltpu.VMEM((_T, D), jnp.float32),
            pltpu.VMEM((_T, D), jnp.float32),
            pltpu.VMEM((_T,), jnp.float32),
            pltpu.VMEM((256,), jnp.float32),
            pltpu.VMEM((D,), jnp.float32),
            pltpu.VMEM((D,), jnp.float32),
            pltpu.VMEM((D,), jnp.float32),
            pltpu.VMEM((16,), jnp.float32),
            pltpu.SemaphoreType.DMA,
        ],
    )
    def p1(tabl, tabr, src, dst, av, e_out, maxp_out,
           sidx_v, didx_v, rows_s, rows_d, e_stage, stage16,
           a_v, a6_v, a4_v, mx_v, sem):
        wid = lax.axis_index("s") * _NC + lax.axis_index("c")
        pltpu.sync_copy(av, a_v)
        for i in range(D // 16):
            sl = pl.ds(i * 16, 16)
            a = a_v[sl]
            a6_v[sl] = a * 0.6
            a4_v[sl] = a * 0.4
        lanes16 = lax.iota(jnp.int32, 16) * 16

        def batch_body(b, mx):
            base = wid * _EW + b * _T
            pltpu.sync_copy(src.at[pl.ds(base, _T)], sidx_v)
            pltpu.sync_copy(dst.at[pl.ds(base, _T)], didx_v)
            pltpu.async_copy(tabl.at[sidx_v], rows_s, sem).wait()
            pltpu.async_copy(tabr.at[didx_v], rows_d, sem).wait()

            def group_body(g, carry):
                gb = g * 16
                for jj in range(16):
                    acc = None
                    for k in range(D // 16):
                        ks = pl.ds(k * 16, 16)
                        t = rows_s[gb + jj, ks] + rows_d[gb + jj, ks]
                        term = a6_v[pl.ds(k * 16, 16)] * t \
                            + a4_v[pl.ds(k * 16, 16)] * jnp.abs(t)
                        acc = term if acc is None else acc + term
                    stage16[pl.ds(jj * 16, 16)] = acc
                tot = None
                for f in range(16):
                    col = plsc.load_gather(stage16, [lanes16 + f])
                    tot = col if tot is None else tot + col
                e_stage[pl.ds(gb, 16)] = tot
                return carry

            lax.fori_loop(0, _T // 16, group_body, 0)
            for g in range(_T // 16):
                mx = jnp.maximum(mx, e_stage[pl.ds(g * 16, 16)])
            pltpu.sync_copy(e_stage, e_out.at[pl.ds(base, _T)])
            return mx

        mx = lax.fori_loop(0, _NB1, batch_body,
                           jnp.full((16,), -3e38, jnp.float32))
        mx_v[...] = mx
        pltpu.sync_copy(mx_v, maxp_out.at[wid])

    return p1


def _build_p2(D):
    CF = D // 128
    NPASS = CF
    T2 = 64
    NB = _EP // T2
    RT = _NH // _NS
    ZB = RT // T2

    @functools.partial(
        pl.kernel,
        out_type=jax.ShapeDtypeStruct((CF, _N, 128), jnp.float32),
        mesh=_sc_mesh(),
        compiler_params=_SC_PARAMS,
        scratch_types=[
            pltpu.VMEM_SHARED((_NH + 16, 128), jnp.float32),
            pltpu.VMEM_SHARED((_NH + 16,), jnp.float32),
            pltpu.VMEM((T2,), jnp.int32),
            pltpu.VMEM((T2,), jnp.int32),
            pltpu.VMEM((T2,), jnp.int32),
            pltpu.VMEM((T2,), jnp.float32),
            pltpu.VMEM((T2,), jnp.float32),
            pltpu.VMEM((T2, 128), jnp.float32),
            pltpu.VMEM((RT,), jnp.float32),
            pltpu.VMEM((_NW, 16), jnp.float32),
            pltpu.VMEM((D,), jnp.float32),
            pltpu.SemaphoreType.DMA,
        ],
    )
    def p2(tab, src, dst, e_in, maxp, cvec, hout,
           acc_s, den_s, sidx_v, didx_v, gidx_v, e_v, ex_v, rows_v,
           den_v, maxp_v, c_v, sem):
        cidx = lax.axis_index("c")
        sidx = lax.axis_index("s")
        pltpu.sync_copy(maxp, maxp_v)
        mx = jnp.full((16,), -3e38, jnp.float32)
        for i in range(_NW):
            mx = jnp.maximum(mx, maxp_v[i])
        m = jnp.max(mx)
        pltpu.sync_copy(cvec, c_v)
        zeros16 = jnp.zeros((16,), jnp.float32)
        trash = jnp.int32(_NH)
        tile0 = sidx * RT

        for p in range(NPASS):
            if CF == 1:
                node_base = cidx * _NH
                fchunk = 0 * cidx
            else:
                node_base = jnp.int32(p * _NH)
                fchunk = cidx

            def zrow_body(r, _):
                for k in range(8):
                    rows_v[r, pl.ds(k * 16, 16)] = zeros16
                return 0

            lax.fori_loop(0, T2, zrow_body, 0)

            def zero_body(r, _):
                pltpu.sync_copy(rows_v, acc_s.at[pl.ds(tile0 + r * T2, T2)])
                return 0

            lax.fori_loop(0, ZB, zero_body, 0)

            def zden_body(r, _):
                pltpu.sync_copy(rows_v.at[0], den_s.at[pl.ds(tile0 + r * 128, 128)])
                return 0

            lax.fori_loop(0, RT // 128, zden_body, 0)
            plsc.subcore_barrier()

            def edge_body(b, _, node_base=node_base):
                base = sidx * _EP + b * T2
                pltpu.sync_copy(src.at[pl.ds(base, T2)], sidx_v)
                pltpu.sync_copy(dst.at[pl.ds(base, T2)], didx_v)
                pltpu.sync_copy(e_in.at[pl.ds(base, T2)], e_v)
                for i in range(T2 // 16):
                    sl = pl.ds(i * 16, 16)
                    if CF == 1:
                        gidx_v[sl] = sidx_v[sl]
                    else:
                        gidx_v[sl] = sidx_v[sl] * CF + cidx
                pltpu.async_copy(tab.at[gidx_v], rows_v, sem).wait()
                for i in range(T2 // 16):
                    sl = pl.ds(i * 16, 16)
                    ex_v[sl] = jnp.exp(e_v[sl] - m)
                    t = didx_v[sl] - node_base
                    ok = jnp.logical_and(t >= 0, t < _NH)
                    didx_v[sl] = jnp.where(ok, t, trash)

                def scale_body(j, _):
                    s = plsc.load_gather(ex_v, [jnp.zeros((16,), jnp.int32) + j])
                    for k in range(8):
                        ks = pl.ds(k * 16, 16)
                        rows_v[j, ks] = rows_v[j, ks] * s
                    return 0

                lax.fori_loop(0, T2, scale_body, 0)
                pltpu.sync_copy(rows_v, acc_s.at[didx_v], add=True)
                pltpu.sync_copy(ex_v, den_s.at[didx_v], add=True)
                return 0

            lax.fori_loop(0, NB, edge_body, 0)
            plsc.subcore_barrier()

            pltpu.sync_copy(den_s.at[pl.ds(tile0, RT)], den_v)

            def drain_body(rb, _, node_base=node_base, fchunk=fchunk):
                row0 = tile0 + rb * T2
                pltpu.sync_copy(acc_s.at[pl.ds(row0, T2)], rows_v)

                def row_body(j, _, rb=rb, fchunk=fchunk):
                    dsp = plsc.load_gather(
                        den_v, [jnp.zeros((16,), jnp.int32) + (rb * T2 + j)])
                    inv = 1.0 / (dsp + 1e-16)
                    for k in range(8):
                        ks = pl.ds(k * 16, 16)
                        csl = c_v[pl.ds(fchunk * 128 + k * 16, 16)]
                        rows_v[j, ks] = rows_v[j, ks] * inv + csl
                    return 0

                lax.fori_loop(0, T2, row_body, 0)
                pltpu.sync_copy(rows_v, hout.at[fchunk, pl.ds(node_base + row0, T2)])
                return 0

            lax.fori_loop(0, ZB, drain_body, 0)
            if p + 1 < NPASS:
                plsc.subcore_barrier()

    return p2


_BN = 1024


def _k1_body(x_ref, wl_ref, bl_ref, wr_ref, br_ref, xl_ref, xr_ref):
    xb = x_ref[...]
    xl_ref[...] = jnp.dot(xb, wl_ref[...],
                          preferred_element_type=jnp.float32) + bl_ref[...]
    xr_ref[...] = jnp.dot(xb, wr_ref[...],
                          preferred_element_type=jnp.float32) + br_ref[...]


def _k1(x, W1l, b1l, W1r, b1r):
    return pl.pallas_call(
        _k1_body,
        grid=(_N // _BN,),
        in_specs=[
            pl.BlockSpec((_BN, _D1), lambda i: (i, 0)),
            pl.BlockSpec((_D1, _D1), lambda i: (0, 0)),
            pl.BlockSpec((1, _D1), lambda i: (0, 0)),
            pl.BlockSpec((_D1, _D1), lambda i: (0, 0)),
            pl.BlockSpec((1, _D1), lambda i: (0, 0)),
        ],
        out_specs=[
            pl.BlockSpec((_BN, _D1), lambda i: (i, 0)),
            pl.BlockSpec((_BN, _D1), lambda i: (i, 0)),
        ],
        out_shape=[
            jax.ShapeDtypeStruct((_N, _D1), jnp.float32),
            jax.ShapeDtypeStruct((_N, _D1), jnp.float32),
        ],
    )(x, W1l, b1l.reshape(1, -1), W1r, b1r.reshape(1, -1))


def _ln_block(h, g, b):
    mu = jnp.mean(h, axis=-1, keepdims=True)
    var = jnp.mean((h - mu) ** 2, axis=-1, keepdims=True)
    return (h - mu) / jnp.sqrt(var + 1e-5) * g + b


def _k2_body(h1_ref, x_ref, wl_ref, bl_ref, wr_ref, br_ref,
             g1_ref, be1_ref, wres_ref, bres_ref, xl_ref, xr_ref, xres_ref):
    h = _ln_block(h1_ref[0], g1_ref[...], be1_ref[...])
    h = jnp.maximum(h, 0.0)
    xl_ref[...] = jnp.dot(h, wl_ref[...],
                          preferred_element_type=jnp.float32) + bl_ref[...]
    xr_ref[...] = jnp.dot(h, wr_ref[...],
                          preferred_element_type=jnp.float32) + br_ref[...]
    xres_ref[...] = (jnp.dot(x_ref[...], wres_ref[...],
                             preferred_element_type=jnp.float32) + bres_ref[...])


def _k2(h1, x, W2l, b2l, W2r, b2r, g1, be1, Wres, bres):
    return pl.pallas_call(
        _k2_body,
        grid=(_N // _BN,),
        in_specs=[
            pl.BlockSpec((1, _BN, _D1), lambda i: (0, i, 0)),
            pl.BlockSpec((_BN, _D1), lambda i: (i, 0)),
            pl.BlockSpec((_D1, _D2), lambda i: (0, 0)),
            pl.BlockSpec((1, _D2), lambda i: (0, 0)),
            pl.BlockSpec((_D1, _D2), lambda i: (0, 0)),
            pl.BlockSpec((1, _D2), lambda i: (0, 0)),
            pl.BlockSpec((1, _D1), lambda i: (0, 0)),
            pl.BlockSpec((1, _D1), lambda i: (0, 0)),
            pl.BlockSpec((_D1, _D2), lambda i: (0, 0)),
            pl.BlockSpec((1, _D2), lambda i: (0, 0)),
        ],
        out_specs=[
            pl.BlockSpec((_BN, _D2), lambda i: (i, 0)),
            pl.BlockSpec((_BN, _D2), lambda i: (i, 0)),
            pl.BlockSpec((_BN, _D2), lambda i: (i, 0)),
        ],
        out_shape=[
            jax.ShapeDtypeStruct((_N, _D2), jnp.float32),
            jax.ShapeDtypeStruct((_N, _D2), jnp.float32),
            jax.ShapeDtypeStruct((_N, _D2), jnp.float32),
        ],
    )(h1, x, W2l, b2l.reshape(1, -1), W2r, b2r.reshape(1, -1),
      g1.reshape(1, -1), be1.reshape(1, -1), Wres, bres.reshape(1, -1))


_BG = 64
_BGR = _BG * _NPG


def _k3_body(h2_ref, xres_ref, g2_ref, be2_ref, m_ref, p_ref, out_ref):
    h = jnp.concatenate([h2_ref[0], h2_ref[1]], axis=-1)
    h = _ln_block(h, g2_ref[...], be2_ref[...])
    h = jnp.tanh(h + xres_ref[...])
    s = lax.dot_general(h, h, (((1,), (1,)), ((), ())),
                        preferred_element_type=jnp.float32) * (1.0 / 16.0)
    ew = jnp.exp(s) * m_ref[...]
    rs = jnp.sum(ew, axis=1, keepdims=True)
    zb = jnp.dot(m_ref[...], rs, preferred_element_type=jnp.float32)
    w = rs / zb
    out_ref[...] = jnp.dot(p_ref[...], h * w, preferred_element_type=jnp.float32)


def _k3(h2, xres, g2, be2, mask, pool):
    return pl.pallas_call(
        _k3_body,
        grid=(_N // _BGR,),
        in_specs=[
            pl.BlockSpec((2, _BGR, 128), lambda i: (0, i, 0)),
            pl.BlockSpec((_BGR, _D2), lambda i: (i, 0)),
            pl.BlockSpec((1, _D2), lambda i: (0, 0)),
            pl.BlockSpec((1, _D2), lambda i: (0, 0)),
            pl.BlockSpec((_BGR, _BGR), lambda i: (0, 0)),
            pl.BlockSpec((_BG, _BGR), lambda i: (0, 0)),
        ],
        out_specs=pl.BlockSpec((_BG, _D2), lambda i: (i, 0)),
        out_shape=jax.ShapeDtypeStruct((_B, _D2), jnp.float32),
    )(h2, xres, g2.reshape(1, -1), be2.reshape(1, -1), mask, pool)


_B4 = 256


def _k4_body(ha_ref, haf_ref, outv_ref, z_ref):
    i = pl.program_id(0)
    s = lax.dot_general(ha_ref[...], haf_ref[...], (((1,), (1,)), ((), ())),
                        preferred_element_type=jnp.float32) * (1.0 / 16.0)
    ew = jnp.exp(s)
    rs = jnp.sum(ew, axis=1, keepdims=True)
    part = jnp.sum(ha_ref[...] * rs, axis=0, keepdims=True)
    zpart = jnp.sum(rs)

    @pl.when(i == 0)
    def _():
        outv_ref[...] = part
        z_ref[0] = zpart

    @pl.when(i > 0)
    def _():
        outv_ref[...] = outv_ref[...] + part
        z_ref[0] = z_ref[0] + zpart

    @pl.when(i == _B // _B4 - 1)
    def _():
        outv_ref[...] = outv_ref[...] / z_ref[0]


def _k4(ha):
    return pl.pallas_call(
        _k4_body,
        grid=(_B // _B4,),
        in_specs=[
            pl.BlockSpec((_B4, _D2), lambda i: (i, 0)),
            pl.BlockSpec((_B, _D2), lambda i: (0, 0)),
        ],
        out_specs=pl.BlockSpec((1, _D2), lambda i: (0, 0)),
        out_shape=jax.ShapeDtypeStruct((1, _D2), jnp.float32),
        scratch_shapes=[pltpu.SMEM((1,), jnp.float32)],
    )(ha, ha)


_P1_L1 = _build_p1(_D1)
_P1_L2 = _build_p1(_D2)
_P2_L1 = _build_p2(_D1)
_P2_L2 = _build_p2(_D2)


def kernel(x, edge_index, batch, W1l, b1l, W1r, b1r, a1, c1, g1, be1,
           W2l, b2l, W2r, b2r, a2, c2, g2, be2, Wres, bres):
    del batch
    sl = jnp.arange(_N, dtype=jnp.int32)
    src = jnp.concatenate([edge_index[0].astype(jnp.int32), sl])
    dst = jnp.concatenate([edge_index[1].astype(jnp.int32), sl])

    gid = jnp.arange(_BGR, dtype=jnp.int32) // _NPG
    mask = (gid[:, None] == gid[None, :]).astype(jnp.float32)
    pool = (jnp.arange(_BG, dtype=jnp.int32)[:, None] == gid[None, :]).astype(jnp.float32)

    xl1, xr1 = _k1(x, W1l, b1l, W1r, b1r)
    e1, mx1 = _P1_L1(xl1, xr1, src, dst, a1)
    h1 = _P2_L1(xl1, src, dst, e1, mx1, c1)

    xl2, xr2, xres = _k2(h1, x, W2l, b2l, W2r, b2r, g1, be1, Wres, bres)
    e2, mx2 = _P1_L2(xl2, xr2, src, dst, a2)
    h2 = _P2_L2(xl2.reshape(2 * _N, 128), src, dst, e2, mx2, c2)

    ha = _k3(h2, xres, g2, be2, mask, pool)
    outv = _k4(ha)
    return outv.reshape(1, 1, _D2)
